# baseline (speedup 1.0000x reference)
import functools

import jax
import jax.numpy as jnp
from jax import lax
from jax.experimental import pallas as pl
from jax.experimental.pallas import tpu as pltpu
from jax.experimental.pallas import tpu_sc as plsc

NC = 2
NS = 16
CH = 128
JUNK = 240


def _sc_mesh():
    return plsc.VectorSubcoreMesh(core_axis_name="c", subcore_axis_name="s")


def _stage_indices(ei_h, idx_v, s, T, ECK, Nn):
    pltpu.sync_copy(ei_h.at[s], idx_v)
    row_lo = jnp.clip(ECK - s * T, 0, T)

    def synth(t, _):
        base = (s * T + t - ECK) * CH
        for j in range(CH // 16):
            ii = base + j * 16 + lax.iota(jnp.int32, 16)
            idx_v[t, pl.ds(j * 16, 16)] = jnp.where(
                ii < Nn, ii, Nn + ii % JUNK)
        return 0

    lax.fori_loop(row_lo, T, synth, 0)


def _deg_call(dst4, NP, T, ECK, Nn):
    RPT = NP // NS
    TH = T // 2

    @functools.partial(
        pl.kernel,
        out_type=jax.ShapeDtypeStruct((NC, NP), jnp.float32),
        mesh=_sc_mesh(),
        compiler_params=pltpu.CompilerParams(use_tc_tiling_on_sc=False),
        scratch_types=[
            pltpu.VMEM((T, CH), jnp.int32),
            pltpu.VMEM((CH,), jnp.float32),
            pltpu.VMEM((RPT,), jnp.float32),
            pltpu.VMEM_SHARED((NP,), jnp.float32),
        ],
    )
    def body(dst_h, out_h, dst_v, ones_v, z_v, deg_sh):
        c = lax.axis_index("c")
        s = lax.axis_index("s")
        _stage_indices(dst_h, dst_v, s, T, ECK, Nn)
        for j in range(CH // 16):
            ones_v[pl.ds(j * 16, 16)] = jnp.ones((16,), jnp.float32)

        def zb(i, _):
            z_v[pl.ds(i * 16, 16)] = jnp.zeros((16,), jnp.float32)
            return 0

        lax.fori_loop(0, RPT // 16, zb, 0)
        pltpu.sync_copy(z_v, deg_sh.at[pl.ds(s * RPT, RPT)])
        plsc.subcore_barrier()

        def eb(t, _):
            pltpu.sync_copy(ones_v, deg_sh.at[dst_v.at[t]], add=True)
            return 0

        lax.fori_loop(c * TH, (c + 1) * TH, eb, 0)
        plsc.subcore_barrier()
        pltpu.sync_copy(deg_sh.at[pl.ds(s * RPT, RPT)],
                        out_h.at[c, pl.ds(s * RPT, RPT)])

    return body(dst4)


def _edge_call(g_lo, g_hi, src4, dst4, zrows, NP, DH, T, ECK, Nn):
    RPT = NP // NS

    @functools.partial(
        pl.kernel,
        out_type=jax.ShapeDtypeStruct((NC, NP, DH), jnp.bfloat16),
        mesh=_sc_mesh(),
        compiler_params=pltpu.CompilerParams(use_tc_tiling_on_sc=False),
        scratch_types=[
            pltpu.VMEM((T, CH), jnp.int32),
            pltpu.VMEM((T, CH), jnp.int32),
            pltpu.VMEM((CH, DH), jnp.bfloat16),
            pltpu.VMEM((CH, DH), jnp.bfloat16),
            pltpu.VMEM((CH, DH), jnp.bfloat16),
            pltpu.VMEM((CH, DH), jnp.bfloat16),
            pltpu.VMEM_SHARED((NP, DH), jnp.bfloat16),
            pltpu.SemaphoreType.DMA,
            pltpu.SemaphoreType.DMA,
            pltpu.SemaphoreType.DMA,
            pltpu.SemaphoreType.DMA,
            pltpu.SemaphoreType.DMA,
            pltpu.SemaphoreType.DMA,
            pltpu.SemaphoreType.DMA,
            pltpu.SemaphoreType.DMA,
        ],
    )
    def body(glo_h, ghi_h, src_h, dst_h, z_h, out_h, src_v, dst_v, rowa, rowb,
             rowc, rowd, acc, gs0, gs1, gs2, gs3, ss0, ss1, ss2, ss3):
        c = lax.axis_index("c")
        s = lax.axis_index("s")
        pltpu.sync_copy(z_h, acc.at[pl.ds(s * RPT, RPT)])
        _stage_indices(src_h, src_v, s, T, ECK, Nn)
        _stage_indices(dst_h, dst_v, s, T, ECK, Nn)
        plsc.subcore_barrier()

        bufs = (rowa, rowb, rowc, rowd)
        gsems = (gs0, gs1, gs2, gs3)
        ssems = (ss0, ss1, ss2, ss3)

        def run(g_h):
            def gstart(t, b):
                pltpu.async_copy(g_h.at[src_v.at[t]], bufs[b], gsems[b])

            def gwait(t, b):
                pltpu.make_async_copy(g_h.at[src_v.at[t]], bufs[b],
                                      gsems[b]).wait()

            def sstart(t, b):
                pltpu.make_async_copy(bufs[b], acc.at[dst_v.at[t]],
                                      ssems[b]).start(add=True)

            def swait(t, b):
                pltpu.make_async_copy(bufs[b], acc.at[dst_v.at[t]],
                                      ssems[b]).wait()

            for k in range(3):
                gstart(k, k)
            for j in range(4):
                if j > 0:
                    swait(j - 1, j - 1)
                gstart(j + 3, (j + 3) % 4)
                gwait(j, j)
                sstart(j, j)

            def group(i, _):
                for j in range(4):
                    t = 4 * i + j
                    swait(t - 1, (j + 3) % 4)
                    gstart(t + 3, (j + 3) % 4)
                    gwait(t, j)
                    sstart(t, j)
                return 0

            lax.fori_loop(1, T // 4 - 1, group, 0)
            tail = 4 * (T // 4 - 1)
            swait(tail - 1, 3)
            gstart(T - 1, 3)
            for j in range(4):
                t = tail + j
                gwait(t, j)
                sstart(t, j)
            for j in range(4):
                swait(tail + j, j)

        @pl.when(c == 0)
        def _():
            run(glo_h)

        @pl.when(c == 1)
        def _():
            run(ghi_h)

        plsc.subcore_barrier()
        pltpu.sync_copy(acc.at[pl.ds(s * RPT, RPT)],
                        out_h.at[c, pl.ds(s * RPT, RPT)])

    return body(g_lo, g_hi, src4, dst4, zrows)


def _tc_mm(inp, W, degt, NP, D):
    BLK = 1024
    DH = D // 2

    def body(x_ref, w_ref, d_ref, glo_ref, ghi_ref):
        h = jnp.dot(x_ref[...], w_ref[...], preferred_element_type=jnp.float32)
        dsum = d_ref[:, 0:1] + d_ref[:, 1:2]
        dis = lax.rsqrt(dsum)
        g = (h * dis).astype(jnp.bfloat16)
        glo_ref[...] = g[:, :DH]
        ghi_ref[...] = g[:, DH:]

    return pl.pallas_call(
        body,
        grid=(NP // BLK,),
        in_specs=[
            pl.BlockSpec((BLK, D), lambda i: (i, 0)),
            pl.BlockSpec((D, D), lambda i: (0, 0)),
            pl.BlockSpec((BLK, NC), lambda i: (i, 0)),
        ],
        out_specs=[
            pl.BlockSpec((BLK, DH), lambda i: (i, 0)),
            pl.BlockSpec((BLK, DH), lambda i: (i, 0)),
        ],
        out_shape=[
            jax.ShapeDtypeStruct((NP, DH), jnp.bfloat16),
            jax.ShapeDtypeStruct((NP, DH), jnp.bfloat16),
        ],
    )(inp, W, degt)


def _tc_fin_mm(acc2, degt, b2d, W, NP, D):
    BLK = 1024
    DH = D // 2

    def body(a_ref, d_ref, b_ref, w_ref, glo_ref, ghi_ref):
        dsum = d_ref[:, 0:1] + d_ref[:, 1:2]
        dis = lax.rsqrt(dsum)
        a = jnp.concatenate([a_ref[0], a_ref[1]],
                            axis=-1).astype(jnp.float32)
        z = jax.nn.sigmoid(dis * a + b_ref[...])
        h2 = jnp.dot(z, w_ref[...], preferred_element_type=jnp.float32)
        g2 = (h2 * dis).astype(jnp.bfloat16)
        glo_ref[...] = g2[:, :DH]
        ghi_ref[...] = g2[:, DH:]

    return pl.pallas_call(
        body,
        grid=(NP // BLK,),
        in_specs=[
            pl.BlockSpec((NC, BLK, DH), lambda i: (0, i, 0)),
            pl.BlockSpec((BLK, NC), lambda i: (i, 0)),
            pl.BlockSpec((1, D), lambda i: (0, 0)),
            pl.BlockSpec((D, D), lambda i: (0, 0)),
        ],
        out_specs=[
            pl.BlockSpec((BLK, DH), lambda i: (i, 0)),
            pl.BlockSpec((BLK, DH), lambda i: (i, 0)),
        ],
        out_shape=[
            jax.ShapeDtypeStruct((NP, DH), jnp.bfloat16),
            jax.ShapeDtypeStruct((NP, DH), jnp.bfloat16),
        ],
    )(acc2, degt, b2d, W)


def _tc_fin(acc2, degt, b2d, N, NP, D):
    BLK = 1024
    DH = D // 2

    def body(a_ref, d_ref, b_ref, z_ref):
        dsum = d_ref[:, 0:1] + d_ref[:, 1:2]
        dis = lax.rsqrt(dsum)
        a = jnp.concatenate([a_ref[0], a_ref[1]],
                            axis=-1).astype(jnp.float32)
        z_ref[...] = jax.nn.sigmoid(dis * a + b_ref[...])

    return pl.pallas_call(
        body,
        grid=(-(-N // BLK),),
        in_specs=[
            pl.BlockSpec((NC, BLK, DH), lambda i: (0, i, 0)),
            pl.BlockSpec((BLK, NC), lambda i: (i, 0)),
            pl.BlockSpec((1, D), lambda i: (0, 0)),
        ],
        out_specs=pl.BlockSpec((BLK, D), lambda i: (i, 0)),
        out_shape=jax.ShapeDtypeStruct((N, D), jnp.float32),
    )(acc2, degt, b2d)


def kernel(x, edge_index, W1, b1, W2, b2):
    N, D = x.shape
    E = edge_index.shape[1]
    DH = D // 2
    NP = N + JUNK
    E2 = E + N
    T = -(-E2 // (NS * CH))
    T += (-T) % 8

    ECK = E // CH
    ei4 = jnp.pad(edge_index.reshape(2, ECK, CH),
                  ((0, 0), (0, NS * T - ECK), (0, 0))).reshape(2, NS, T, CH)
    src4 = ei4[0]
    dst4 = ei4[1]
    zrows = jnp.zeros((NP // NS, DH), jnp.bfloat16)
    b1r = b1.reshape(1, D)
    b2r = b2.reshape(1, D)

    deg2 = _deg_call(dst4, NP, T, ECK, N)
    degt = deg2.T

    g1lo, g1hi = _tc_mm(x, W1, degt, NP, D)
    acc1 = _edge_call(g1lo, g1hi, src4, dst4, zrows, NP, DH, T, ECK, N)
    g2lo, g2hi = _tc_fin_mm(acc1, degt, b1r, W2, NP, D)
    acc2 = _edge_call(g2lo, g2hi, src4, dst4, zrows, NP, DH, T, ECK, N)
    return _tc_fin(acc2, degt, b2r, N, NP, D)


# speedup vs baseline: 1.7835x; 1.7835x over previous
import functools

import numpy as np

import jax
import jax.numpy as jnp
from jax import lax
from jax.experimental import pallas as pl
from jax.experimental.pallas import tpu as pltpu
from jax.experimental.pallas import tpu_sc as plsc

NC = 2
NS = 16
CH = 128
JUNK = 240


def _sc_mesh():
    return plsc.VectorSubcoreMesh(core_axis_name="c", subcore_axis_name="s")


def _deg_call(dst4, NP, T, ECK, Nn):
    RPT = NP // NS
    TH = T // 2

    @functools.partial(
        pl.kernel,
        out_type=jax.ShapeDtypeStruct((NC, NP), jnp.float32),
        mesh=_sc_mesh(),
        compiler_params=pltpu.CompilerParams(use_tc_tiling_on_sc=False),
        scratch_types=[
            pltpu.VMEM((T, CH), jnp.int32),
            pltpu.VMEM((CH,), jnp.float32),
            pltpu.VMEM((RPT,), jnp.float32),
            pltpu.VMEM_SHARED((NP,), jnp.float32),
        ],
    )
    def body(dst_h, out_h, dst_v, ones_v, z_v, deg_sh):
        c = lax.axis_index("c")
        s = lax.axis_index("s")
        pltpu.sync_copy(dst_h.at[s], dst_v)
        for j in range(CH // 16):
            ones_v[pl.ds(j * 16, 16)] = jnp.ones((16,), jnp.float32)

        def zb(i, _):
            z_v[pl.ds(i * 16, 16)] = jnp.zeros((16,), jnp.float32)
            return 0

        lax.fori_loop(0, RPT // 16, zb, 0)
        pltpu.sync_copy(z_v, deg_sh.at[pl.ds(s * RPT, RPT)])
        plsc.subcore_barrier()

        def eb(t, _):
            pltpu.sync_copy(ones_v, deg_sh.at[dst_v.at[t]], add=True)
            return 0

        lax.fori_loop(c * TH, (c + 1) * TH, eb, 0)
        plsc.subcore_barrier()
        pltpu.sync_copy(deg_sh.at[pl.ds(s * RPT, RPT)],
                        out_h.at[c, pl.ds(s * RPT, RPT)])

    return body(dst4)


def _edge_call(g_lo, g_hi, src4, dst4, zrows, NP, DH, T, ECK, Nn):
    RPT = NP // NS

    @functools.partial(
        pl.kernel,
        out_type=jax.ShapeDtypeStruct((NC, NP, DH), jnp.bfloat16),
        mesh=_sc_mesh(),
        compiler_params=pltpu.CompilerParams(use_tc_tiling_on_sc=False),
        scratch_types=[
            pltpu.VMEM((T, CH), jnp.int32),
            pltpu.VMEM((T, CH), jnp.int32),
            pltpu.VMEM((CH, DH), jnp.bfloat16),
            pltpu.VMEM((CH, DH), jnp.bfloat16),
            pltpu.VMEM((CH, DH), jnp.bfloat16),
            pltpu.VMEM((CH, DH), jnp.bfloat16),
            pltpu.VMEM_SHARED((NP, DH), jnp.bfloat16),
            pltpu.SemaphoreType.DMA,
            pltpu.SemaphoreType.DMA,
            pltpu.SemaphoreType.DMA,
            pltpu.SemaphoreType.DMA,
            pltpu.SemaphoreType.DMA,
            pltpu.SemaphoreType.DMA,
            pltpu.SemaphoreType.DMA,
            pltpu.SemaphoreType.DMA,
        ],
    )
    def body(glo_h, ghi_h, src_h, dst_h, z_h, out_h, src_v, dst_v, rowa, rowb,
             rowc, rowd, acc, gs0, gs1, gs2, gs3, ss0, ss1, ss2, ss3):
        c = lax.axis_index("c")
        s = lax.axis_index("s")
        pltpu.sync_copy(z_h, acc.at[pl.ds(s * RPT, RPT)])
        pltpu.sync_copy(src_h.at[s], src_v)
        pltpu.sync_copy(dst_h.at[s], dst_v)
        plsc.subcore_barrier()

        bufs = (rowa, rowb, rowc, rowd)
        gsems = (gs0, gs1, gs2, gs3)
        ssems = (ss0, ss1, ss2, ss3)

        def run(g_h):
            def gstart(t, b):
                pltpu.async_copy(g_h.at[src_v.at[t]], bufs[b], gsems[b])

            def gwait(t, b):
                pltpu.make_async_copy(g_h.at[src_v.at[t]], bufs[b],
                                      gsems[b]).wait()

            def sstart(t, b):
                pltpu.make_async_copy(bufs[b], acc.at[dst_v.at[t]],
                                      ssems[b]).start(add=True)

            def swait(t, b):
                pltpu.make_async_copy(bufs[b], acc.at[dst_v.at[t]],
                                      ssems[b]).wait()

            for k in range(3):
                gstart(k, k)
            for j in range(4):
                if j > 0:
                    swait(j - 1, j - 1)
                gstart(j + 3, (j + 3) % 4)
                gwait(j, j)
                sstart(j, j)

            def group(i, _):
                for j in range(4):
                    t = 4 * i + j
                    swait(t - 1, (j + 3) % 4)
                    gstart(t + 3, (j + 3) % 4)
                    gwait(t, j)
                    sstart(t, j)
                return 0

            lax.fori_loop(1, T // 4 - 1, group, 0)
            tail = 4 * (T // 4 - 1)
            swait(tail - 1, 3)
            gstart(T - 1, 3)
            for j in range(4):
                t = tail + j
                gwait(t, j)
                sstart(t, j)
            for j in range(4):
                swait(tail + j, j)

        @pl.when(c == 0)
        def _():
            run(glo_h)

        @pl.when(c == 1)
        def _():
            run(ghi_h)

        plsc.subcore_barrier()
        pltpu.sync_copy(acc.at[pl.ds(s * RPT, RPT)],
                        out_h.at[c, pl.ds(s * RPT, RPT)])

    return body(g_lo, g_hi, src4, dst4, zrows)


def _tc_mm(inp, W, degt, NP, D):
    BLK = 1024
    DH = D // 2

    def body(x_ref, w_ref, d_ref, glo_ref, ghi_ref):
        h = jnp.dot(x_ref[...], w_ref[...], preferred_element_type=jnp.float32)
        dsum = d_ref[:, 0:1] + d_ref[:, 1:2]
        dis = lax.rsqrt(dsum)
        g = (h * dis).astype(jnp.bfloat16)
        glo_ref[...] = g[:, :DH]
        ghi_ref[...] = g[:, DH:]

    return pl.pallas_call(
        body,
        grid=(NP // BLK,),
        in_specs=[
            pl.BlockSpec((BLK, D), lambda i: (i, 0)),
            pl.BlockSpec((D, D), lambda i: (0, 0)),
            pl.BlockSpec((BLK, NC), lambda i: (i, 0)),
        ],
        out_specs=[
            pl.BlockSpec((BLK, DH), lambda i: (i, 0)),
            pl.BlockSpec((BLK, DH), lambda i: (i, 0)),
        ],
        out_shape=[
            jax.ShapeDtypeStruct((NP, DH), jnp.bfloat16),
            jax.ShapeDtypeStruct((NP, DH), jnp.bfloat16),
        ],
    )(inp, W, degt)


def _tc_fin_mm(acc2, degt, b2d, W, NP, D):
    BLK = 1024
    DH = D // 2

    def body(a_ref, d_ref, b_ref, w_ref, glo_ref, ghi_ref):
        dsum = d_ref[:, 0:1] + d_ref[:, 1:2]
        dis = lax.rsqrt(dsum)
        a = jnp.concatenate([a_ref[0], a_ref[1]],
                            axis=-1).astype(jnp.float32)
        z = jax.nn.sigmoid(dis * a + b_ref[...])
        h2 = jnp.dot(z, w_ref[...], preferred_element_type=jnp.float32)
        g2 = (h2 * dis).astype(jnp.bfloat16)
        glo_ref[...] = g2[:, :DH]
        ghi_ref[...] = g2[:, DH:]

    return pl.pallas_call(
        body,
        grid=(NP // BLK,),
        in_specs=[
            pl.BlockSpec((NC, BLK, DH), lambda i: (0, i, 0)),
            pl.BlockSpec((BLK, NC), lambda i: (i, 0)),
            pl.BlockSpec((1, D), lambda i: (0, 0)),
            pl.BlockSpec((D, D), lambda i: (0, 0)),
        ],
        out_specs=[
            pl.BlockSpec((BLK, DH), lambda i: (i, 0)),
            pl.BlockSpec((BLK, DH), lambda i: (i, 0)),
        ],
        out_shape=[
            jax.ShapeDtypeStruct((NP, DH), jnp.bfloat16),
            jax.ShapeDtypeStruct((NP, DH), jnp.bfloat16),
        ],
    )(acc2, degt, b2d, W)


def _tc_fin(acc2, degt, b2d, N, NP, D):
    BLK = 1024
    DH = D // 2

    def body(a_ref, d_ref, b_ref, z_ref):
        dsum = d_ref[:, 0:1] + d_ref[:, 1:2]
        dis = lax.rsqrt(dsum)
        a = jnp.concatenate([a_ref[0], a_ref[1]],
                            axis=-1).astype(jnp.float32)
        z_ref[...] = jax.nn.sigmoid(dis * a + b_ref[...])

    return pl.pallas_call(
        body,
        grid=(-(-N // BLK),),
        in_specs=[
            pl.BlockSpec((NC, BLK, DH), lambda i: (0, i, 0)),
            pl.BlockSpec((BLK, NC), lambda i: (i, 0)),
            pl.BlockSpec((1, D), lambda i: (0, 0)),
        ],
        out_specs=pl.BlockSpec((BLK, D), lambda i: (i, 0)),
        out_shape=jax.ShapeDtypeStruct((N, D), jnp.float32),
    )(acc2, degt, b2d)


def kernel(x, edge_index, W1, b1, W2, b2):
    N, D = x.shape
    E = edge_index.shape[1]
    DH = D // 2
    NP = N + JUNK
    E2 = E + N
    T = -(-E2 // (NS * CH))
    T += (-T) % 4

    ECK = E // CH
    NCK = NS * T
    ti = np.arange((NCK - ECK) * CH, dtype=np.int64)
    tail = np.where(ti < N, ti, N + ti % JUNK).astype(np.int32)
    tail3 = jnp.asarray(np.broadcast_to(
        tail.reshape(1, NCK - ECK, CH), (2, NCK - ECK, CH)))
    ei4 = jnp.concatenate([edge_index.reshape(2, ECK, CH), tail3],
                          axis=1).reshape(2, NS, T, CH)
    src4 = ei4[0]
    dst4 = ei4[1]
    zrows = jnp.zeros((NP // NS, DH), jnp.bfloat16)
    b1r = b1.reshape(1, D)
    b2r = b2.reshape(1, D)

    deg2 = _deg_call(dst4, NP, T, ECK, N)
    degt = deg2.T

    g1lo, g1hi = _tc_mm(x, W1, degt, NP, D)
    acc1 = _edge_call(g1lo, g1hi, src4, dst4, zrows, NP, DH, T, ECK, N)
    g2lo, g2hi = _tc_fin_mm(acc1, degt, b1r, W2, NP, D)
    acc2 = _edge_call(g2lo, g2hi, src4, dst4, zrows, NP, DH, T, ECK, N)
    return _tc_fin(acc2, degt, b2r, N, NP, D)


# speedup vs baseline: 1.7875x; 1.0023x over previous
import functools

import numpy as np

import jax
import jax.numpy as jnp
from jax import lax
from jax.experimental import pallas as pl
from jax.experimental.pallas import tpu as pltpu
from jax.experimental.pallas import tpu_sc as plsc

NC = 2
NS = 16
CH = 128
JUNK = 240


def _sc_mesh():
    return plsc.VectorSubcoreMesh(core_axis_name="c", subcore_axis_name="s")


def _deg_call(dst4, NP, T, ECK, Nn):
    RPT = NP // NS
    TH = T // 2

    @functools.partial(
        pl.kernel,
        out_type=jax.ShapeDtypeStruct((NC, NP), jnp.float32),
        mesh=_sc_mesh(),
        compiler_params=pltpu.CompilerParams(use_tc_tiling_on_sc=False),
        scratch_types=[
            pltpu.VMEM((T, CH), jnp.int32),
            pltpu.VMEM((CH,), jnp.float32),
            pltpu.VMEM((RPT,), jnp.float32),
            pltpu.VMEM_SHARED((NP,), jnp.float32),
        ],
    )
    def body(dst_h, out_h, dst_v, ones_v, z_v, deg_sh):
        c = lax.axis_index("c")
        s = lax.axis_index("s")
        pltpu.sync_copy(dst_h.at[s], dst_v)
        for j in range(CH // 16):
            ones_v[pl.ds(j * 16, 16)] = jnp.ones((16,), jnp.float32)

        def zb(i, _):
            z_v[pl.ds(i * 16, 16)] = jnp.zeros((16,), jnp.float32)
            return 0

        lax.fori_loop(0, RPT // 16, zb, 0)
        pltpu.sync_copy(z_v, deg_sh.at[pl.ds(s * RPT, RPT)])
        plsc.subcore_barrier()

        def eb(t, _):
            pltpu.sync_copy(ones_v, deg_sh.at[dst_v.at[t]], add=True)
            return 0

        lax.fori_loop(c * TH, (c + 1) * TH, eb, 0)
        plsc.subcore_barrier()
        pltpu.sync_copy(deg_sh.at[pl.ds(s * RPT, RPT)],
                        out_h.at[c, pl.ds(s * RPT, RPT)])

    return body(dst4)


def _edge_call(g2c, src4, dst4, zrows, NP, DH, T, ECK, Nn):
    RPT = NP // NS

    @functools.partial(
        pl.kernel,
        out_type=jax.ShapeDtypeStruct((NC, NP, DH), jnp.bfloat16),
        mesh=_sc_mesh(),
        compiler_params=pltpu.CompilerParams(use_tc_tiling_on_sc=False),
        scratch_types=[
            pltpu.VMEM((T, CH), jnp.int32),
            pltpu.VMEM((T, CH), jnp.int32),
            pltpu.VMEM((CH, DH), jnp.bfloat16),
            pltpu.VMEM((CH, DH), jnp.bfloat16),
            pltpu.VMEM((CH, DH), jnp.bfloat16),
            pltpu.VMEM((CH, DH), jnp.bfloat16),
            pltpu.VMEM_SHARED((NP, DH), jnp.bfloat16),
            pltpu.SemaphoreType.DMA,
            pltpu.SemaphoreType.DMA,
            pltpu.SemaphoreType.DMA,
            pltpu.SemaphoreType.DMA,
            pltpu.SemaphoreType.DMA,
            pltpu.SemaphoreType.DMA,
            pltpu.SemaphoreType.DMA,
            pltpu.SemaphoreType.DMA,
        ],
    )
    def body(g_h2, src_h, dst_h, z_h, out_h, src_v, dst_v, rowa, rowb,
             rowc, rowd, acc, gs0, gs1, gs2, gs3, ss0, ss1, ss2, ss3):
        c = lax.axis_index("c")
        s = lax.axis_index("s")
        pltpu.sync_copy(z_h, acc.at[pl.ds(s * RPT, RPT)])
        pltpu.sync_copy(src_h.at[s], src_v)
        pltpu.sync_copy(dst_h.at[s], dst_v)
        plsc.subcore_barrier()

        bufs = (rowa, rowb, rowc, rowd)
        gsems = (gs0, gs1, gs2, gs3)
        ssems = (ss0, ss1, ss2, ss3)

        def run(g_h):
            def gstart(t, b):
                pltpu.async_copy(g_h.at[src_v.at[t]], bufs[b], gsems[b])

            def gwait(t, b):
                pltpu.make_async_copy(g_h.at[src_v.at[t]], bufs[b],
                                      gsems[b]).wait()

            def sstart(t, b):
                pltpu.make_async_copy(bufs[b], acc.at[dst_v.at[t]],
                                      ssems[b]).start(add=True)

            def swait(t, b):
                pltpu.make_async_copy(bufs[b], acc.at[dst_v.at[t]],
                                      ssems[b]).wait()

            for k in range(3):
                gstart(k, k)
            for j in range(4):
                if j > 0:
                    swait(j - 1, j - 1)
                gstart(j + 3, (j + 3) % 4)
                gwait(j, j)
                sstart(j, j)

            def group(i, _):
                for j in range(4):
                    t = 4 * i + j
                    swait(t - 1, (j + 3) % 4)
                    gstart(t + 3, (j + 3) % 4)
                    gwait(t, j)
                    sstart(t, j)
                return 0

            lax.fori_loop(1, T // 4 - 1, group, 0)
            tail = 4 * (T // 4 - 1)
            swait(tail - 1, 3)
            gstart(T - 1, 3)
            for j in range(4):
                t = tail + j
                gwait(t, j)
                sstart(t, j)
            for j in range(4):
                swait(tail + j, j)

        @pl.when(c == 0)
        def _():
            run(g_h2.at[0])

        @pl.when(c == 1)
        def _():
            run(g_h2.at[1])

        plsc.subcore_barrier()
        pltpu.sync_copy(acc.at[pl.ds(s * RPT, RPT)],
                        out_h.at[c, pl.ds(s * RPT, RPT)])

    return body(g2c, src4, dst4, zrows)


def _tc_mm(inp, W, degt, NP, D):
    BLK = 1024
    DH = D // 2

    def body(x_ref, w_ref, d_ref, g_ref):
        h = jnp.dot(x_ref[...], w_ref[...], preferred_element_type=jnp.float32)
        dsum = d_ref[:, 0:1] + d_ref[:, 1:2]
        dis = lax.rsqrt(dsum)
        g = (h * dis).astype(jnp.bfloat16)
        g_ref[0] = g[:, :DH]
        g_ref[1] = g[:, DH:]

    return pl.pallas_call(
        body,
        grid=(NP // BLK,),
        in_specs=[
            pl.BlockSpec((BLK, D), lambda i: (i, 0)),
            pl.BlockSpec((D, D), lambda i: (0, 0)),
            pl.BlockSpec((BLK, NC), lambda i: (i, 0)),
        ],
        out_specs=pl.BlockSpec((NC, BLK, DH), lambda i: (0, i, 0)),
        out_shape=jax.ShapeDtypeStruct((NC, NP, DH), jnp.bfloat16),
    )(inp, W, degt)


def _tc_fin_mm(acc2, degt, b2d, W, NP, D):
    BLK = 1024
    DH = D // 2

    def body(a_ref, d_ref, b_ref, w_ref, g_ref):
        dsum = d_ref[:, 0:1] + d_ref[:, 1:2]
        dis = lax.rsqrt(dsum)
        a = jnp.concatenate([a_ref[0], a_ref[1]],
                            axis=-1).astype(jnp.float32)
        z = jax.nn.sigmoid(dis * a + b_ref[...])
        h2 = jnp.dot(z, w_ref[...], preferred_element_type=jnp.float32)
        g2 = (h2 * dis).astype(jnp.bfloat16)
        g_ref[0] = g2[:, :DH]
        g_ref[1] = g2[:, DH:]

    return pl.pallas_call(
        body,
        grid=(NP // BLK,),
        in_specs=[
            pl.BlockSpec((NC, BLK, DH), lambda i: (0, i, 0)),
            pl.BlockSpec((BLK, NC), lambda i: (i, 0)),
            pl.BlockSpec((1, D), lambda i: (0, 0)),
            pl.BlockSpec((D, D), lambda i: (0, 0)),
        ],
        out_specs=pl.BlockSpec((NC, BLK, DH), lambda i: (0, i, 0)),
        out_shape=jax.ShapeDtypeStruct((NC, NP, DH), jnp.bfloat16),
    )(acc2, degt, b2d, W)


def _tc_fin(acc2, degt, b2d, N, NP, D):
    BLK = 1024
    DH = D // 2

    def body(a_ref, d_ref, b_ref, z_ref):
        dsum = d_ref[:, 0:1] + d_ref[:, 1:2]
        dis = lax.rsqrt(dsum)
        a = jnp.concatenate([a_ref[0], a_ref[1]],
                            axis=-1).astype(jnp.float32)
        z_ref[...] = jax.nn.sigmoid(dis * a + b_ref[...])

    return pl.pallas_call(
        body,
        grid=(-(-N // BLK),),
        in_specs=[
            pl.BlockSpec((NC, BLK, DH), lambda i: (0, i, 0)),
            pl.BlockSpec((BLK, NC), lambda i: (i, 0)),
            pl.BlockSpec((1, D), lambda i: (0, 0)),
        ],
        out_specs=pl.BlockSpec((BLK, D), lambda i: (i, 0)),
        out_shape=jax.ShapeDtypeStruct((N, D), jnp.float32),
    )(acc2, degt, b2d)


def kernel(x, edge_index, W1, b1, W2, b2):
    N, D = x.shape
    E = edge_index.shape[1]
    DH = D // 2
    NP = N + JUNK
    E2 = E + N
    T = -(-E2 // (NS * CH))
    T += (-T) % 4

    ECK = E // CH
    NCK = NS * T
    ti = np.arange((NCK - ECK) * CH, dtype=np.int64)
    tail = np.where(ti < N, ti, N + ti % JUNK).astype(np.int32)
    tail3 = jnp.asarray(np.broadcast_to(
        tail.reshape(1, NCK - ECK, CH), (2, NCK - ECK, CH)))
    ei4 = jnp.concatenate([edge_index.reshape(2, ECK, CH), tail3],
                          axis=1).reshape(2, NS, T, CH)
    src4 = ei4[0]
    dst4 = ei4[1]
    zrows = jnp.zeros((NP // NS, DH), jnp.bfloat16)
    b1r = b1.reshape(1, D)
    b2r = b2.reshape(1, D)

    deg2 = _deg_call(dst4, NP, T, ECK, N)
    degt = deg2.T

    g1 = _tc_mm(x, W1, degt, NP, D)
    acc1 = _edge_call(g1, src4, dst4, zrows, NP, DH, T, ECK, N)
    g2 = _tc_fin_mm(acc1, degt, b1r, W2, NP, D)
    acc2 = _edge_call(g2, src4, dst4, zrows, NP, DH, T, ECK, N)
    return _tc_fin(acc2, degt, b2r, N, NP, D)
